# baseline (speedup 1.0000x reference)
import functools

import jax
import jax.numpy as jnp
from jax import lax
from jax.experimental import pallas as pl
from jax.experimental.pallas import tpu as pltpu
from jax.experimental.pallas import tpu_sc as plsc

B = 16384
L = 20
NNEG = 5
D = 64
PD = 128
NC, NS, LANES = 2, 16, 16
NW = NC * NS
EPW = B // NW
CB = 32
NBLK = EPW // CB
TN = 1 + NNEG
SLOTS = 16
PAD_SCORE = 1e4


def _sc_scores(ctx_flat, ctx32_flat, tn_flat, syn0p, syn1p):
    mesh = plsc.VectorSubcoreMesh(core_axis_name="c", subcore_axis_name="s")

    @functools.partial(
        pl.kernel,
        out_type=jax.ShapeDtypeStruct((B * SLOTS,), jnp.float32),
        mesh=mesh,
        compiler_params=pltpu.CompilerParams(
            needs_layout_passes=False, use_tc_tiling_on_sc=True),
        scratch_types=[
            pltpu.VMEM((CB * L,), jnp.int32),
            pltpu.VMEM((CB * L,), jnp.int32),
            pltpu.VMEM((CB * 2 * LANES,), jnp.int32),
            pltpu.VMEM((CB * TN + LANES,), jnp.int32),
            pltpu.VMEM((CB * TN,), jnp.int32),
            pltpu.VMEM((CB * L, PD), jnp.float32),
            pltpu.VMEM((CB * TN, PD), jnp.float32),
            pltpu.VMEM((CB * SLOTS,), jnp.float32),
            pltpu.SemaphoreType.DMA,
        ],
    )
    def k(ctx_hbm, ctx32_hbm, tn_hbm, syn0_hbm, syn1_hbm, out_hbm,
          idx_ctx, idxp_ctx, idx32, idx_tn, idxp_tn, rows_ctx, rows_tn,
          scores, sem):
        wid = lax.axis_index("s") * NC + lax.axis_index("c")
        lane = lax.iota(jnp.int32, LANES)

        def block(g, carry):
            base = wid * EPW + g * CB
            pltpu.sync_copy(ctx_hbm.at[pl.ds(base * L, CB * L)], idx_ctx)
            pltpu.sync_copy(ctx32_hbm.at[pl.ds(base * 2 * LANES, CB * 2 * LANES)], idx32)
            pltpu.sync_copy(tn_hbm.at[pl.ds(base * TN, CB * TN)],
                            idx_tn.at[pl.ds(0, CB * TN)])
            for t in range(CB * L // LANES):
                idxp_ctx[pl.ds(t * LANES, LANES)] = (
                    idx_ctx[pl.ds(t * LANES, LANES)] >> 1)
            for t in range(CB * TN // LANES):
                idxp_tn[pl.ds(t * LANES, LANES)] = (
                    idx_tn[pl.ds(t * LANES, LANES)] >> 1)
            handles = []
            for t in range(CB * L // 128):
                handles.append(pltpu.async_copy(
                    syn0_hbm.at[idxp_ctx.at[pl.ds(t * 128, 128)]],
                    rows_ctx.at[pl.ds(t * 128, 128)], sem))
            for t in range(2):
                half = CB * TN // 2
                handles.append(pltpu.async_copy(
                    syn1_hbm.at[idxp_tn.at[pl.ds(t * half, half)]],
                    rows_tn.at[pl.ds(t * half, half)], sem))
            for h in handles:
                h.wait()

            def elem(e, carry2):
                v1 = idx32[pl.ds(e * 2 * LANES, LANES)]
                v2 = idx32[pl.ds(e * 2 * LANES + LANES, LANES)]
                cnt = (jnp.sum(jnp.where(v1 != 0, 1.0, 0.0))
                       + jnp.sum(jnp.where(v2 != 0, 1.0, 0.0)))
                rcp = 1.0 / jnp.full((LANES,), cnt, jnp.float32)
                pv1 = (idx_ctx[pl.ds(e * L, LANES)] & 1) * D
                pv2 = (idx_ctx[pl.ds(e * L + (L - LANES), LANES)] & 1) * D
                offs = [pv1[r] for r in range(LANES)]
                offs += [pv2[r] for r in range(2 * LANES - L, LANES)]
                mean = [None] * 4
                for r in range(L):
                    for c in range(4):
                        v = rows_ctx[e * L + r, pl.ds(offs[r] + c * LANES, LANES)]
                        mean[c] = v if r == 0 else mean[c] + v
                mean = [m * rcp for m in mean]
                tv = (idx_tn[pl.ds(e * TN, LANES)] & 1) * D
                s = jnp.full((LANES,), PAD_SCORE, jnp.float32)
                for n in range(TN):
                    off = tv[n]
                    acc = mean[0] * rows_tn[e * TN + n, pl.ds(off, LANES)]
                    for c in range(1, 4):
                        acc = acc + mean[c] * rows_tn[
                            e * TN + n, pl.ds(off + c * LANES, LANES)]
                    val = jnp.sum(acc) if n == 0 else -jnp.sum(acc)
                    s = jnp.where(lane == n, val, s)
                scores[pl.ds(e * SLOTS, SLOTS)] = s
                return carry2

            lax.fori_loop(0, CB, elem, 0)
            pltpu.sync_copy(scores, out_hbm.at[pl.ds(base * SLOTS, CB * SLOTS)])
            return carry

        lax.fori_loop(0, NBLK, block, 0)

    return k(ctx_flat, ctx32_flat, tn_flat, syn0p, syn1p)


def _tc_loss(scores2d):
    def body(s_ref, o_ref):
        x = s_ref[...]
        ls = jnp.minimum(x, 0.0) - jnp.log1p(jnp.exp(-jnp.abs(x)))
        o_ref[...] = jnp.full((1, 1), -jnp.sum(ls), jnp.float32)

    out = pl.pallas_call(
        body,
        out_shape=jax.ShapeDtypeStruct((1, 1), jnp.float32),
    )(scores2d)
    return out[0, 0]


def kernel(target, context, negatives, syn0, syn1):
    ctx_flat = context.reshape(-1).astype(jnp.int32)
    ctx32 = jnp.pad(context.astype(jnp.int32), ((0, 0), (0, 2 * LANES - L)))
    tn = jnp.concatenate([target[:, None].astype(jnp.int32),
                          negatives.astype(jnp.int32)], axis=1)
    syn0p = syn0.reshape(-1, PD)
    syn1p = syn1.reshape(-1, PD)
    scores = _sc_scores(ctx_flat, ctx32.reshape(-1), tn.reshape(-1),
                        syn0p, syn1p)
    return _tc_loss(scores.reshape(B * SLOTS // 128, 128))


# speedup vs baseline: 1.4572x; 1.4572x over previous
import functools

import jax
import jax.numpy as jnp
from jax import lax
from jax.experimental import pallas as pl
from jax.experimental.pallas import tpu as pltpu
from jax.experimental.pallas import tpu_sc as plsc

B = 16384
L = 20
NNEG = 5
D = 64
NC, NS, LANES = 2, 16, 16
NW = NC * NS
EPW = B // NW
CB = 32
NBLK = EPW // CB
TN = 1 + NNEG
SLOTS = 16
PAD_SCORE = 1e4


def _sc_scores(ctx_flat, ctx32_flat, tn_flat, syn0, syn1):
    mesh = plsc.VectorSubcoreMesh(core_axis_name="c", subcore_axis_name="s")

    @functools.partial(
        pl.kernel,
        out_type=jax.ShapeDtypeStruct((B * SLOTS,), jnp.float32),
        mesh=mesh,
        compiler_params=pltpu.CompilerParams(
            needs_layout_passes=False, use_tc_tiling_on_sc=True),
        scratch_types=[
            pltpu.VMEM((CB * L,), jnp.int32),
            pltpu.VMEM((CB * 2 * LANES,), jnp.int32),
            pltpu.VMEM((CB * TN,), jnp.int32),
            pltpu.VMEM((CB * L, D), jnp.float32),
            pltpu.VMEM((CB * TN, D), jnp.float32),
            pltpu.VMEM((CB * SLOTS,), jnp.float32),
            pltpu.SemaphoreType.DMA,
        ],
    )
    def k(ctx_hbm, ctx32_hbm, tn_hbm, syn0_hbm, syn1_hbm, out_hbm,
          idx_ctx, idx32, idx_tn, rows_ctx, rows_tn, scores, sem):
        wid = lax.axis_index("s") * NC + lax.axis_index("c")
        lane = lax.iota(jnp.int32, LANES)

        def block(g, carry):
            base = wid * EPW + g * CB
            pltpu.sync_copy(ctx_hbm.at[pl.ds(base * L, CB * L)], idx_ctx)
            pltpu.sync_copy(ctx32_hbm.at[pl.ds(base * 2 * LANES, CB * 2 * LANES)], idx32)
            pltpu.sync_copy(tn_hbm.at[pl.ds(base * TN, CB * TN)], idx_tn)

            def enq_ctx(t, carry2):
                iv = idx_ctx[pl.ds(t * LANES, LANES)]
                for j in range(LANES):
                    pltpu.async_copy(syn0_hbm.at[iv[j]],
                                     rows_ctx.at[t * LANES + j], sem)
                return carry2

            lax.fori_loop(0, CB * L // LANES, enq_ctx, 0)

            def enq_tn(t, carry2):
                iv = idx_tn[pl.ds(t * LANES, LANES)]
                for j in range(LANES):
                    pltpu.async_copy(syn1_hbm.at[iv[j]],
                                     rows_tn.at[t * LANES + j], sem)
                return carry2

            lax.fori_loop(0, CB * TN // LANES, enq_tn, 0)

            pltpu.make_async_copy(syn0_hbm.at[pl.ds(0, CB * L)], rows_ctx,
                                  sem).wait()
            pltpu.make_async_copy(syn1_hbm.at[pl.ds(0, CB * TN)], rows_tn,
                                  sem).wait()

            def elem(e, carry2):
                v1 = idx32[pl.ds(e * 2 * LANES, LANES)]
                v2 = idx32[pl.ds(e * 2 * LANES + LANES, LANES)]
                cnt = (jnp.sum(jnp.where(v1 != 0, 1.0, 0.0))
                       + jnp.sum(jnp.where(v2 != 0, 1.0, 0.0)))
                rcp = 1.0 / jnp.full((LANES,), cnt, jnp.float32)
                mean = [None] * 4
                for r in range(L):
                    for c in range(4):
                        v = rows_ctx[e * L + r, pl.ds(c * LANES, LANES)]
                        mean[c] = v if r == 0 else mean[c] + v
                mean = [m * rcp for m in mean]
                s = jnp.full((LANES,), PAD_SCORE, jnp.float32)
                for n in range(TN):
                    acc = mean[0] * rows_tn[e * TN + n, pl.ds(0, LANES)]
                    for c in range(1, 4):
                        acc = acc + mean[c] * rows_tn[
                            e * TN + n, pl.ds(c * LANES, LANES)]
                    val = jnp.sum(acc) if n == 0 else -jnp.sum(acc)
                    s = jnp.where(lane == n, val, s)
                scores[pl.ds(e * SLOTS, SLOTS)] = s
                return carry2

            lax.fori_loop(0, CB, elem, 0)
            pltpu.sync_copy(scores, out_hbm.at[pl.ds(base * SLOTS, CB * SLOTS)])
            return carry

        lax.fori_loop(0, NBLK, block, 0)

    return k(ctx_flat, ctx32_flat, tn_flat, syn0, syn1)


def _tc_loss(scores2d):
    def body(s_ref, o_ref):
        x = s_ref[...]
        ls = jnp.minimum(x, 0.0) - jnp.log1p(jnp.exp(-jnp.abs(x)))
        o_ref[...] = jnp.full((1, 1), -jnp.sum(ls), jnp.float32)

    out = pl.pallas_call(
        body,
        out_shape=jax.ShapeDtypeStruct((1, 1), jnp.float32),
    )(scores2d)
    return out[0, 0]


def kernel(target, context, negatives, syn0, syn1):
    ctx_flat = context.reshape(-1).astype(jnp.int32)
    ctx32 = jnp.pad(context.astype(jnp.int32), ((0, 0), (0, 2 * LANES - L)))
    tn = jnp.concatenate([target[:, None].astype(jnp.int32),
                          negatives.astype(jnp.int32)], axis=1)
    scores = _sc_scores(ctx_flat, ctx32.reshape(-1), tn.reshape(-1),
                        syn0, syn1)
    return _tc_loss(scores.reshape(B * SLOTS // 128, 128))
